# Initial kernel scaffold; baseline (speedup 1.0000x reference)
import jax
import jax.numpy as jnp
from jax.experimental import pallas as pl


def kernel(x, edge_index, W1, b1, W2, b2):
    raise NotImplementedError("write your pallas kernel here")


# speedup vs baseline: 37.7361x; 37.7361x over previous
import functools

import jax
import jax.numpy as jnp
from jax import lax
from jax.experimental import pallas as pl
from jax.experimental.pallas import tpu as pltpu
from jax.experimental.pallas import tpu_sc as plsc

N = 10000
D_IN = 128
D_HID = 64
D_OUT2 = 8
E = 320000

NC = 2
NS = 16
CH = 128
CPT_SUM = 157
NCHT = NS * CPT_SUM
E_PAD = NCHT * CH
NPAD = 10240
ROWS_PT = NPAD // NS

SPLIT_DEG = (79, 78)
SPLIT1 = (105, 52)
SPLIT2 = (90, 67)

_MESH = plsc.VectorSubcoreMesh(core_axis_name="c", subcore_axis_name="s")
_SC_PARAMS = pltpu.CompilerParams(use_tc_tiling_on_sc=False)


def _sc_degree(dst2, ones_h, zeros_h):
    cpt_max = max(SPLIT_DEG)

    @functools.partial(
        pl.kernel,
        out_type=jax.ShapeDtypeStruct((NC, NPAD), jnp.float32),
        mesh=_MESH,
        scratch_types=[
            pltpu.VMEM((cpt_max, CH), jnp.int32),
            pltpu.VMEM((CH,), jnp.float32),
            pltpu.VMEM_SHARED((NPAD,), jnp.float32),
        ],
        compiler_params=_SC_PARAMS,
    )
    def k(dst_h, ones_hbm, zeros_hbm, parts, didx, ones_v, dacc):
        c = lax.axis_index("c")
        s = lax.axis_index("s")
        pltpu.sync_copy(zeros_hbm, dacc.at[pl.ds(s * ROWS_PT, ROWS_PT)])
        pltpu.sync_copy(ones_hbm, ones_v)

        for ci in range(NC):
            cpt = SPLIT_DEG[ci]
            base = 0 if ci == 0 else NS * SPLIT_DEG[0]

            @pl.when(c == ci)
            def _(cpt=cpt, base=base):
                start = base + s * cpt
                pltpu.sync_copy(dst_h.at[pl.ds(start, cpt)],
                                didx.at[pl.ds(0, cpt)])
                plsc.subcore_barrier()

                @pl.loop(0, cpt)
                def _(j):
                    pltpu.sync_copy(ones_v, dacc.at[didx.at[j]], add=True)

        plsc.subcore_barrier()
        pltpu.sync_copy(
            dacc.at[pl.ds(s * ROWS_PT, ROWS_PT)],
            parts.at[c, pl.ds(s * ROWS_PT, ROWS_PT)],
        )

    return k(dst2, ones_h, zeros_h)


def _sc_aggregate(table, src2, dst2, zeros_h, width, split):
    cpt_max = max(split)

    @functools.partial(
        pl.kernel,
        out_type=jax.ShapeDtypeStruct((NC, NPAD, width), jnp.float32),
        mesh=_MESH,
        scratch_types=[
            pltpu.VMEM((cpt_max, CH), jnp.int32),
            pltpu.VMEM((cpt_max, CH), jnp.int32),
            [pltpu.VMEM((CH, width), jnp.float32) for _ in range(4)],
            pltpu.VMEM_SHARED((NPAD, width), jnp.float32),
            [pltpu.SemaphoreType.DMA for _ in range(4)],
        ],
        compiler_params=_SC_PARAMS,
    )
    def k(tab_h, src_h, dst_h, zeros_hbm, parts,
          sidx, didx, rows, acc, sems):
        c = lax.axis_index("c")
        s = lax.axis_index("s")

        @pl.loop(0, ROWS_PT // CH)
        def _(kk):
            pltpu.sync_copy(zeros_hbm, acc.at[pl.ds(s * ROWS_PT + kk * CH, CH)])

        for ci in range(NC):
            cpt = split[ci]
            base = 0 if ci == 0 else NS * split[0]

            @pl.when(c == ci)
            def _(cpt=cpt, base=base):
                start = base + s * cpt
                pltpu.sync_copy(src_h.at[pl.ds(start, cpt)],
                                sidx.at[pl.ds(0, cpt)])
                pltpu.sync_copy(dst_h.at[pl.ds(start, cpt)],
                                didx.at[pl.ds(0, cpt)])
                plsc.subcore_barrier()

                for j in range(3):
                    pltpu.make_async_copy(
                        tab_h.at[sidx.at[j]], rows[j], sems[j]).start()

                @pl.loop(0, (cpt + 3) // 4)
                def _(i):
                    for b in range(4):
                        j = 4 * i + b
                        nxt = j + 3
                        bn = (b + 3) % 4

                        @pl.when(nxt < cpt)
                        def _():
                            pltpu.make_async_copy(
                                tab_h.at[sidx.at[nxt]], rows[bn], sems[bn]
                            ).start()

                        @pl.when(j < cpt)
                        def _():
                            pltpu.make_async_copy(
                                tab_h.at[sidx.at[j]], rows[b], sems[b]
                            ).wait()
                            pltpu.sync_copy(rows[b], acc.at[didx.at[j]],
                                            add=True)

        plsc.subcore_barrier()

        @pl.loop(0, ROWS_PT // CH)
        def _(kk):
            r = s * ROWS_PT + kk * CH
            pltpu.sync_copy(acc.at[pl.ds(r, CH)], parts.at[c, pl.ds(r, CH)])

    return k(table, src2, dst2, zeros_h)


def _tc_matmul1(x, W1):
    def body(x_ref, w_ref, o_ref):
        o_ref[...] = jnp.dot(x_ref[...], w_ref[...],
                             preferred_element_type=jnp.float32)

    return pl.pallas_call(
        body, out_shape=jax.ShapeDtypeStruct((N, D_HID), jnp.float32)
    )(x, W1)


def _tc_scale(h1, d0, d1):
    def body(h_ref, d0_ref, d1_ref, g_ref, dinv_ref):
        deg = d0_ref[...] + d1_ref[...] + 1.0
        dinv = lax.rsqrt(deg)
        dinv_ref[...] = dinv
        g_ref[...] = h_ref[...] * dinv

    return pl.pallas_call(
        body,
        out_shape=[
            jax.ShapeDtypeStruct((N, D_HID), jnp.float32),
            jax.ShapeDtypeStruct((N, 1), jnp.float32),
        ],
    )(h1, d0, d1)


def _tc_layer2_in(p0, p1, g1, dinv, b1r, w2p):
    def body(p0_ref, p1_ref, g1_ref, dinv_ref, b1_ref, w2_ref, g2_ref):
        s1 = p0_ref[...] + p1_ref[...] + g1_ref[...]
        a1 = jnp.maximum(s1 * dinv_ref[...] + b1_ref[...], 0.0)
        h2 = jnp.dot(a1, w2_ref[...], preferred_element_type=jnp.float32)
        g2_ref[...] = h2 * dinv_ref[...]

    return pl.pallas_call(
        body, out_shape=jax.ShapeDtypeStruct((N, D_OUT2), jnp.float32)
    )(p0, p1, g1, dinv, b1r, w2p)


def _tc_final(p0, p1, g2, dinv, b2r):
    def body(p0_ref, p1_ref, g2_ref, dinv_ref, b2_ref, o_ref):
        o_ref[...] = ((p0_ref[...] + p1_ref[...] + g2_ref[...])
                      * dinv_ref[...] + b2_ref[...])

    return pl.pallas_call(
        body, out_shape=jax.ShapeDtypeStruct((N, D_OUT2), jnp.float32)
    )(p0, p1, g2, dinv, b2r)


def kernel(x, edge_index, W1, b1, W2, b2):
    pad = E_PAD - E
    src2 = jnp.concatenate(
        [edge_index[0], jnp.zeros((pad,), jnp.int32)]).reshape(NCHT, CH)
    dst2 = jnp.concatenate(
        [edge_index[1], jnp.full((pad,), N, jnp.int32)]).reshape(NCHT, CH)

    ones_h = jnp.ones((CH,), jnp.float32)
    zeros_deg = jnp.zeros((ROWS_PT,), jnp.float32)
    zeros64 = jnp.zeros((CH, D_HID), jnp.float32)
    zeros8 = jnp.zeros((CH, D_OUT2), jnp.float32)
    b1r = b1.reshape(1, D_HID)
    b2r = jnp.pad(b2, (0, D_OUT2 - b2.shape[0])).reshape(1, D_OUT2)
    w2p = jnp.pad(W2, ((0, 0), (0, D_OUT2 - W2.shape[1])))

    deg_parts = _sc_degree(dst2, ones_h, zeros_deg)
    h1 = _tc_matmul1(x, W1)

    dp = deg_parts[:, :N, None]
    g1, dinv = _tc_scale(h1, dp[0], dp[1])

    parts1 = _sc_aggregate(g1, src2, dst2, zeros64, D_HID, SPLIT1)
    g2 = _tc_layer2_in(parts1[0, :N], parts1[1, :N], g1, dinv, b1r, w2p)

    parts2 = _sc_aggregate(g2, src2, dst2, zeros8, D_OUT2, SPLIT2)
    out16 = _tc_final(parts2[0, :N], parts2[1, :N], g2, dinv, b2r)
    return out16[:, :2]


# speedup vs baseline: 39.8017x; 1.0547x over previous
import functools

import jax
import jax.numpy as jnp
from jax import lax
from jax.experimental import pallas as pl
from jax.experimental.pallas import tpu as pltpu
from jax.experimental.pallas import tpu_sc as plsc

N = 10000
D_IN = 128
D_HID = 64
D_OUT2 = 8
E = 320000

NC = 2
NS = 16
CH = 128
CPT_SUM = 157
NCHT = NS * CPT_SUM
E_PAD = NCHT * CH
NPAD = 10240
ROWS_PT = NPAD // NS

SPLIT_DEG = (79, 78)
SPLIT1 = (110, 47)
SPLIT2 = (88, 69)

BN = 1000
NB = N // BN

_MESH = plsc.VectorSubcoreMesh(core_axis_name="c", subcore_axis_name="s")
_SC_PARAMS = pltpu.CompilerParams(use_tc_tiling_on_sc=False)


def _sc_degree(dst2, ones_h, zeros_h):
    cpt_max = max(SPLIT_DEG)

    @functools.partial(
        pl.kernel,
        out_type=jax.ShapeDtypeStruct((NC, NPAD, 1), jnp.float32),
        mesh=_MESH,
        scratch_types=[
            pltpu.VMEM((cpt_max, CH), jnp.int32),
            pltpu.VMEM((CH, 1), jnp.float32),
            pltpu.VMEM_SHARED((NPAD, 1), jnp.float32),
        ],
        compiler_params=_SC_PARAMS,
    )
    def k(dst_h, ones_hbm, zeros_hbm, parts, didx, ones_v, dacc):
        c = lax.axis_index("c")
        s = lax.axis_index("s")
        pltpu.sync_copy(zeros_hbm, dacc.at[pl.ds(s * ROWS_PT, ROWS_PT)])
        pltpu.sync_copy(ones_hbm, ones_v)

        for ci in range(NC):
            cpt = SPLIT_DEG[ci]
            base = 0 if ci == 0 else NS * SPLIT_DEG[0]

            @pl.when(c == ci)
            def _(cpt=cpt, base=base):
                start = base + s * cpt
                pltpu.sync_copy(dst_h.at[pl.ds(start, cpt)],
                                didx.at[pl.ds(0, cpt)])
                plsc.subcore_barrier()

                @pl.loop(0, cpt)
                def _(j):
                    pltpu.sync_copy(ones_v, dacc.at[didx.at[j]], add=True)

        plsc.subcore_barrier()
        pltpu.sync_copy(
            dacc.at[pl.ds(s * ROWS_PT, ROWS_PT)],
            parts.at[c, pl.ds(s * ROWS_PT, ROWS_PT)],
        )

    return k(dst2, ones_h, zeros_h)


def _sc_aggregate(table, src2, dst2, zeros_h, width, split):
    cpt_max = max(split)

    @functools.partial(
        pl.kernel,
        out_type=jax.ShapeDtypeStruct((NC, NPAD, width), jnp.float32),
        mesh=_MESH,
        scratch_types=[
            pltpu.VMEM((cpt_max, CH), jnp.int32),
            pltpu.VMEM((cpt_max, CH), jnp.int32),
            [pltpu.VMEM((CH, width), jnp.float32) for _ in range(4)],
            pltpu.VMEM_SHARED((NPAD, width), jnp.float32),
            [pltpu.SemaphoreType.DMA for _ in range(4)],
        ],
        compiler_params=_SC_PARAMS,
    )
    def k(tab_h, src_h, dst_h, zeros_hbm, parts,
          sidx, didx, rows, acc, sems):
        c = lax.axis_index("c")
        s = lax.axis_index("s")

        @pl.loop(0, ROWS_PT // CH)
        def _(kk):
            pltpu.sync_copy(zeros_hbm, acc.at[pl.ds(s * ROWS_PT + kk * CH, CH)])

        for ci in range(NC):
            cpt = split[ci]
            base = 0 if ci == 0 else NS * split[0]

            @pl.when(c == ci)
            def _(cpt=cpt, base=base):
                start = base + s * cpt
                pltpu.sync_copy(src_h.at[pl.ds(start, cpt)],
                                sidx.at[pl.ds(0, cpt)])
                pltpu.sync_copy(dst_h.at[pl.ds(start, cpt)],
                                didx.at[pl.ds(0, cpt)])
                plsc.subcore_barrier()

                for j in range(3):
                    pltpu.make_async_copy(
                        tab_h.at[sidx.at[j]], rows[j], sems[j]).start()

                @pl.loop(0, (cpt + 3) // 4)
                def _(i):
                    for b in range(4):
                        j = 4 * i + b
                        nxt = j + 3
                        bn = (b + 3) % 4

                        @pl.when(nxt < cpt)
                        def _():
                            pltpu.make_async_copy(
                                tab_h.at[sidx.at[nxt]], rows[bn], sems[bn]
                            ).start()

                        @pl.when(j < cpt)
                        def _():
                            pltpu.make_async_copy(
                                tab_h.at[sidx.at[j]], rows[b], sems[b]
                            ).wait()
                            pltpu.sync_copy(rows[b], acc.at[didx.at[j]],
                                            add=True)

        plsc.subcore_barrier()

        @pl.loop(0, ROWS_PT // CH)
        def _(kk):
            r = s * ROWS_PT + kk * CH
            pltpu.sync_copy(acc.at[pl.ds(r, CH)], parts.at[c, pl.ds(r, CH)])

    return k(table, src2, dst2, zeros_h)


def _row_spec(d):
    return pl.BlockSpec((BN, d), lambda i: (i, 0))


def _parts_spec(d):
    return pl.BlockSpec((NC, BN, d), lambda i: (0, i, 0))


def _fixed_spec(r, d):
    return pl.BlockSpec((r, d), lambda i: (0, 0))


def _tc_matmul1(x, W1):
    def body(x_ref, w_ref, o_ref):
        o_ref[...] = jnp.dot(x_ref[...], w_ref[...],
                             preferred_element_type=jnp.float32)

    return pl.pallas_call(
        body,
        grid=(NB,),
        in_specs=[_row_spec(D_IN), _fixed_spec(D_IN, D_HID)],
        out_specs=_row_spec(D_HID),
        out_shape=jax.ShapeDtypeStruct((N, D_HID), jnp.float32),
    )(x, W1)


def _tc_scale(h1, deg_parts):
    def body(h_ref, dp_ref, g_ref, dinv_ref):
        deg = dp_ref[0] + dp_ref[1] + 1.0
        dinv = lax.rsqrt(deg)
        dinv_ref[...] = dinv
        g_ref[...] = h_ref[...] * dinv

    return pl.pallas_call(
        body,
        grid=(NB,),
        in_specs=[_row_spec(D_HID), _parts_spec(1)],
        out_specs=[_row_spec(D_HID), _row_spec(1)],
        out_shape=[
            jax.ShapeDtypeStruct((N, D_HID), jnp.float32),
            jax.ShapeDtypeStruct((N, 1), jnp.float32),
        ],
    )(h1, deg_parts)


def _tc_layer2_in(parts1, g1, dinv, b1r, w2p):
    def body(p_ref, g1_ref, dinv_ref, b1_ref, w2_ref, g2_ref):
        s1 = p_ref[0] + p_ref[1] + g1_ref[...]
        a1 = jnp.maximum(s1 * dinv_ref[...] + b1_ref[...], 0.0)
        h2 = jnp.dot(a1, w2_ref[...], preferred_element_type=jnp.float32)
        g2_ref[...] = h2 * dinv_ref[...]

    return pl.pallas_call(
        body,
        grid=(NB,),
        in_specs=[_parts_spec(D_HID), _row_spec(D_HID), _row_spec(1),
                  _fixed_spec(1, D_HID), _fixed_spec(D_HID, D_OUT2)],
        out_specs=_row_spec(D_OUT2),
        out_shape=jax.ShapeDtypeStruct((N, D_OUT2), jnp.float32),
    )(parts1, g1, dinv, b1r, w2p)


def _tc_final(parts2, g2, dinv, b2r):
    def body(p_ref, g2_ref, dinv_ref, b2_ref, o_ref):
        res = (p_ref[0] + p_ref[1] + g2_ref[...]) * dinv_ref[...] + b2_ref[...]
        o_ref[...] = res[:, :2]

    return pl.pallas_call(
        body,
        grid=(NB,),
        in_specs=[_parts_spec(D_OUT2), _row_spec(D_OUT2), _row_spec(1),
                  _fixed_spec(1, D_OUT2)],
        out_specs=_row_spec(2),
        out_shape=jax.ShapeDtypeStruct((N, 2), jnp.float32),
    )(parts2, g2, dinv, b2r)


def kernel(x, edge_index, W1, b1, W2, b2):
    pad = E_PAD - E
    src2 = jnp.concatenate(
        [edge_index[0], jnp.zeros((pad,), jnp.int32)]).reshape(NCHT, CH)
    dst2 = jnp.concatenate(
        [edge_index[1], jnp.full((pad,), N, jnp.int32)]).reshape(NCHT, CH)

    ones_h = jnp.ones((CH, 1), jnp.float32)
    zeros_deg = jnp.zeros((ROWS_PT, 1), jnp.float32)
    zeros64 = jnp.zeros((CH, D_HID), jnp.float32)
    zeros8 = jnp.zeros((CH, D_OUT2), jnp.float32)
    b1r = b1.reshape(1, D_HID)
    b2r = jnp.pad(b2, (0, D_OUT2 - b2.shape[0])).reshape(1, D_OUT2)
    w2p = jnp.pad(W2, ((0, 0), (0, D_OUT2 - W2.shape[1])))

    deg_parts = _sc_degree(dst2, ones_h, zeros_deg)
    h1 = _tc_matmul1(x, W1)

    g1, dinv = _tc_scale(h1, deg_parts)

    parts1 = _sc_aggregate(g1, src2, dst2, zeros64, D_HID, SPLIT1)
    g2 = _tc_layer2_in(parts1, g1, dinv, b1r, w2p)

    parts2 = _sc_aggregate(g2, src2, dst2, zeros8, D_OUT2, SPLIT2)
    return _tc_final(parts2, g2, dinv, b2r)


# speedup vs baseline: 41.9411x; 1.0538x over previous
import functools

import jax
import jax.numpy as jnp
from jax import lax
from jax.experimental import pallas as pl
from jax.experimental.pallas import tpu as pltpu
from jax.experimental.pallas import tpu_sc as plsc

N = 10000
D_IN = 128
D_HID = 64
D_OUT2 = 8
E = 320000

NC = 2
NS = 16
CH = 128
NCH = E // CH
NPAD = 10240
ROWS_PT = NPAD // NS

CPT0_DEG = 79
CPT0_1 = 110
CPT0_2 = 88


def _c1_split(cpt0):
    rem = NCH - NS * cpt0
    lo = rem // NS
    n_hi = rem - NS * lo
    return lo, n_hi

_MESH = plsc.VectorSubcoreMesh(core_axis_name="c", subcore_axis_name="s")
_SC_PARAMS = pltpu.CompilerParams(use_tc_tiling_on_sc=False)


def _sc_degree(dst2, ones_h, zeros_h):
    lo, n_hi = _c1_split(CPT0_DEG)

    @functools.partial(
        pl.kernel,
        out_type=jax.ShapeDtypeStruct((NC, NPAD, 1), jnp.float32),
        mesh=_MESH,
        scratch_types=[
            pltpu.VMEM((CPT0_DEG, CH), jnp.int32),
            pltpu.VMEM((CH, 1), jnp.float32),
            pltpu.VMEM_SHARED((NPAD, 1), jnp.float32),
        ],
        compiler_params=_SC_PARAMS,
    )
    def k(dst_h, ones_hbm, zeros_hbm, parts, didx, ones_v, dacc):
        c = lax.axis_index("c")
        s = lax.axis_index("s")
        pltpu.sync_copy(zeros_hbm, dacc.at[pl.ds(s * ROWS_PT, ROWS_PT)])
        pltpu.sync_copy(ones_hbm, ones_v)

        def body(cpt, start):
            pltpu.sync_copy(dst_h.at[pl.ds(start, cpt)],
                            didx.at[pl.ds(0, cpt)])
            plsc.subcore_barrier()

            @pl.loop(0, cpt)
            def _(j):
                pltpu.sync_copy(ones_v, dacc.at[didx.at[j]], add=True)

        @pl.when(c == 0)
        def _():
            body(CPT0_DEG, s * CPT0_DEG)

        @pl.when(jnp.logical_and(c == 1, s < n_hi))
        def _():
            body(lo + 1, NS * CPT0_DEG + s * (lo + 1))

        @pl.when(jnp.logical_and(c == 1, s >= n_hi))
        def _():
            body(lo, NS * CPT0_DEG + n_hi * (lo + 1) + (s - n_hi) * lo)

        plsc.subcore_barrier()
        pltpu.sync_copy(
            dacc.at[pl.ds(s * ROWS_PT, ROWS_PT)],
            parts.at[c, pl.ds(s * ROWS_PT, ROWS_PT)],
        )

    return k(dst2, ones_h, zeros_h)


def _sc_aggregate(table, src2, dst2, zeros_h, width, cpt0):
    lo, n_hi = _c1_split(cpt0)

    @functools.partial(
        pl.kernel,
        out_type=jax.ShapeDtypeStruct((NC, NPAD, width), jnp.float32),
        mesh=_MESH,
        scratch_types=[
            pltpu.VMEM((cpt0, CH), jnp.int32),
            pltpu.VMEM((cpt0, CH), jnp.int32),
            [pltpu.VMEM((CH, width), jnp.float32) for _ in range(4)],
            pltpu.VMEM_SHARED((NPAD, width), jnp.float32),
            [pltpu.SemaphoreType.DMA for _ in range(4)],
        ],
        compiler_params=_SC_PARAMS,
    )
    def k(tab_h, src_h, dst_h, zeros_hbm, parts,
          sidx, didx, rows, acc, sems):
        c = lax.axis_index("c")
        s = lax.axis_index("s")

        @pl.loop(0, ROWS_PT // CH)
        def _(kk):
            pltpu.sync_copy(zeros_hbm, acc.at[pl.ds(s * ROWS_PT + kk * CH, CH)])

        def body(cpt, start):
            pltpu.sync_copy(src_h.at[pl.ds(start, cpt)],
                            sidx.at[pl.ds(0, cpt)])
            pltpu.sync_copy(dst_h.at[pl.ds(start, cpt)],
                            didx.at[pl.ds(0, cpt)])
            plsc.subcore_barrier()

            for j in range(3):
                pltpu.make_async_copy(
                    tab_h.at[sidx.at[j]], rows[j], sems[j]).start()

            @pl.loop(0, (cpt + 3) // 4)
            def _(i):
                for b in range(4):
                    j = 4 * i + b
                    nxt = j + 3
                    bn = (b + 3) % 4

                    @pl.when(nxt < cpt)
                    def _():
                        pltpu.make_async_copy(
                            tab_h.at[sidx.at[nxt]], rows[bn], sems[bn]
                        ).start()

                    @pl.when(j < cpt)
                    def _():
                        pltpu.make_async_copy(
                            tab_h.at[sidx.at[j]], rows[b], sems[b]
                        ).wait()
                        pltpu.sync_copy(rows[b], acc.at[didx.at[j]],
                                        add=True)

        @pl.when(c == 0)
        def _():
            body(cpt0, s * cpt0)

        @pl.when(jnp.logical_and(c == 1, s < n_hi))
        def _():
            body(lo + 1, NS * cpt0 + s * (lo + 1))

        @pl.when(jnp.logical_and(c == 1, s >= n_hi))
        def _():
            body(lo, NS * cpt0 + n_hi * (lo + 1) + (s - n_hi) * lo)

        plsc.subcore_barrier()

        @pl.loop(0, ROWS_PT // CH)
        def _(kk):
            r = s * ROWS_PT + kk * CH
            pltpu.sync_copy(acc.at[pl.ds(r, CH)], parts.at[c, pl.ds(r, CH)])

    return k(table, src2, dst2, zeros_h)


def _tc_matmul1(x, W1):
    def body(x_ref, w_ref, o_ref):
        o_ref[...] = jnp.dot(x_ref[...], w_ref[...],
                             preferred_element_type=jnp.float32)

    return pl.pallas_call(
        body, out_shape=jax.ShapeDtypeStruct((N, D_HID), jnp.float32)
    )(x, W1)


def _tc_scale(h1, deg_parts):
    def body(h_ref, dp_ref, g_ref, dinv_ref):
        deg = dp_ref[0, :N] + dp_ref[1, :N] + 1.0
        dinv = lax.rsqrt(deg)
        dinv_ref[...] = dinv
        g_ref[...] = h_ref[...] * dinv

    return pl.pallas_call(
        body,
        out_shape=[
            jax.ShapeDtypeStruct((N, D_HID), jnp.float32),
            jax.ShapeDtypeStruct((N, 1), jnp.float32),
        ],
    )(h1, deg_parts)


def _tc_layer2_in(parts1, g1, dinv, b1r, w2p):
    def body(p_ref, g1_ref, dinv_ref, b1_ref, w2_ref, g2_ref):
        s1 = p_ref[0, :N] + p_ref[1, :N] + g1_ref[...]
        a1 = jnp.maximum(s1 * dinv_ref[...] + b1_ref[...], 0.0)
        h2 = jnp.dot(a1, w2_ref[...], preferred_element_type=jnp.float32)
        g2_ref[...] = h2 * dinv_ref[...]

    return pl.pallas_call(
        body, out_shape=jax.ShapeDtypeStruct((N, D_OUT2), jnp.float32)
    )(parts1, g1, dinv, b1r, w2p)


def _tc_final(parts2, g2, dinv, b2r):
    def body(p_ref, g2_ref, dinv_ref, b2_ref, o_ref):
        res = ((p_ref[0, :N] + p_ref[1, :N] + g2_ref[...])
               * dinv_ref[...] + b2_ref[...])
        o_ref[...] = res[:, :2]

    return pl.pallas_call(
        body, out_shape=jax.ShapeDtypeStruct((N, 2), jnp.float32)
    )(parts2, g2, dinv, b2r)


def kernel(x, edge_index, W1, b1, W2, b2):
    src2 = edge_index[0].reshape(NCH, CH)
    dst2 = edge_index[1].reshape(NCH, CH)

    ones_h = jnp.ones((CH, 1), jnp.float32)
    zeros_deg = jnp.zeros((ROWS_PT, 1), jnp.float32)
    zeros64 = jnp.zeros((CH, D_HID), jnp.float32)
    zeros8 = jnp.zeros((CH, D_OUT2), jnp.float32)
    b1r = b1.reshape(1, D_HID)
    b2r = jnp.pad(b2, (0, D_OUT2 - b2.shape[0])).reshape(1, D_OUT2)
    w2p = jnp.pad(W2, ((0, 0), (0, D_OUT2 - W2.shape[1])))

    deg_parts = _sc_degree(dst2, ones_h, zeros_deg)
    h1 = _tc_matmul1(x, W1)

    g1, dinv = _tc_scale(h1, deg_parts)

    parts1 = _sc_aggregate(g1, src2, dst2, zeros64, D_HID, CPT0_1)
    g2 = _tc_layer2_in(parts1, g1, dinv, b1r, w2p)

    parts2 = _sc_aggregate(g2, src2, dst2, zeros8, D_OUT2, CPT0_2)
    return _tc_final(parts2, g2, dinv, b2r)
